# baseline (speedup 1.0000x reference)
import numpy as np
import jax
import jax.numpy as jnp
from jax.experimental import pallas as pl
from jax.experimental.pallas import tpu as pltpu

CH, IT, TR, ST = 64, 32, 32, 12
L = 50
NID = 11
NX = NID + ST
OW = CH + 3 * IT + 7 * TR + ST
C = 128

_SLOT_ROWS = [3, 3, 3, 7, 7, 7, 7, 7, 7, 7]
_SLOT_OFF = [CH + 32 * i for i in range(10)]
_BIG = np.float32(1e30)


def _consts():
    E = np.zeros((NX, C), np.float32)
    K = np.full((C,), _BIG, np.float32)
    K2 = np.full((C,), _BIG, np.float32)
    K[0], K2[0] = -_BIG, _BIG
    for s in range(10):
        nr = _SLOT_ROWS[s]
        for k in range(8):
            j = 1 + s * 8 + k
            if k < nr:
                E[1 + s, j] = 1.0
                K[j] = -_BIG if k == 0 else np.float32(k)
                K2[j] = _BIG if k == nr - 1 else np.float32(k + 1)
    return jnp.asarray(E), jnp.asarray(K), jnp.asarray(K2)


def _mixmat(champion_w, item_w, trait_w):
    M = jnp.zeros((C, OW), jnp.float32)
    M = M.at[0, :CH].set(champion_w[0])
    for s in range(10):
        tab = item_w if s < 3 else trait_w
        nr = _SLOT_ROWS[s]
        off = _SLOT_OFF[s]
        M = M.at[1 + s * 8:1 + s * 8 + nr, off:off + 32].set(tab)
    return M


def _body(x_ref, e_ref, k_ref, k2_ref, m_ref, o_ref):
    x = x_ref[...]
    idsf = jnp.clip(jnp.floor(x), 0.0, 7.0)
    spread = jax.lax.dot_general(
        idsf, e_ref[...],
        dimension_numbers=(((2,), (0,)), ((), ())),
        preferred_element_type=jnp.float32,
    )
    k = k_ref[...].reshape(1, 1, C)
    k2 = k2_ref[...].reshape(1, 1, C)
    hot = jnp.where((spread >= k) & (spread < k2), 1.0, 0.0)
    emb = jax.lax.dot_general(
        hot, m_ref[...],
        dimension_numbers=(((2,), (0,)), ((), ())),
        preferred_element_type=jnp.float32,
    )
    o_ref[...] = emb
    o_ref[:, :, CH + 320:] = x[:, :, NID:]


def kernel(x, champion_w, item_w, trait_w):
    B = x.shape[0]
    rb = 128
    E, K, K2 = _consts()
    M = _mixmat(champion_w, item_w, trait_w)
    return pl.pallas_call(
        _body,
        grid=(B // rb,),
        in_specs=[
            pl.BlockSpec((rb, L, NX), lambda i: (i, 0, 0)),
            pl.BlockSpec((NX, C), lambda i: (0, 0)),
            pl.BlockSpec((C,), lambda i: (0,)),
            pl.BlockSpec((C,), lambda i: (0,)),
            pl.BlockSpec((C, OW), lambda i: (0, 0)),
        ],
        out_specs=pl.BlockSpec((rb, L, OW), lambda i: (i, 0, 0)),
        out_shape=jax.ShapeDtypeStruct((B, L, OW), x.dtype),
        compiler_params=pltpu.CompilerParams(
            dimension_semantics=("arbitrary",),
        ),
    )(x, E, K, K2, M)


# speedup vs baseline: 1.0004x; 1.0004x over previous
import numpy as np
import jax
import jax.numpy as jnp
from jax.experimental import pallas as pl
from jax.experimental.pallas import tpu as pltpu

CH, IT, TR, ST = 64, 32, 32, 12
L = 50
NID = 11
NX = NID + ST
OW = CH + 3 * IT + 7 * TR + ST
C = 128

_SLOT_ROWS = [3, 3, 3, 7, 7, 7, 7, 7, 7, 7]
_SLOT_OFF = [CH + 32 * i for i in range(10)]
_BIG = np.float32(1e30)


def _consts():
    E = np.zeros((NX, C), np.float32)
    K = np.full((C,), _BIG, np.float32)
    K2 = np.full((C,), _BIG, np.float32)
    K[0], K2[0] = -_BIG, _BIG
    for s in range(10):
        nr = _SLOT_ROWS[s]
        for k in range(8):
            j = 1 + s * 8 + k
            if k < nr:
                E[1 + s, j] = 1.0
                K[j] = -_BIG if k == 0 else np.float32(k)
                K2[j] = _BIG if k == nr - 1 else np.float32(k + 1)
    return jnp.asarray(E), jnp.asarray(K), jnp.asarray(K2)


def _mixmat(champion_w, item_w, trait_w):
    M = jnp.zeros((C, OW), jnp.float32)
    M = M.at[0, :CH].set(champion_w[0])
    for s in range(10):
        tab = item_w if s < 3 else trait_w
        nr = _SLOT_ROWS[s]
        off = _SLOT_OFF[s]
        M = M.at[1 + s * 8:1 + s * 8 + nr, off:off + 32].set(tab)
    return M


def _body(x_ref, e_ref, k_ref, k2_ref, m_ref, o_ref):
    x = x_ref[...]
    idsf = jnp.clip(jnp.floor(x), 0.0, 7.0)
    spread = jax.lax.dot_general(
        idsf, e_ref[...],
        dimension_numbers=(((2,), (0,)), ((), ())),
        preferred_element_type=jnp.float32,
    )
    k = k_ref[...].reshape(1, 1, C)
    k2 = k2_ref[...].reshape(1, 1, C)
    hot = jnp.where((spread >= k) & (spread < k2), 1.0, 0.0)
    emb = jax.lax.dot_general(
        hot, m_ref[...],
        dimension_numbers=(((2,), (0,)), ((), ())),
        preferred_element_type=jnp.float32,
    )
    o_ref[...] = emb
    o_ref[:, :, CH + 320:] = x[:, :, NID:]


def kernel(x, champion_w, item_w, trait_w):
    B = x.shape[0]
    rb = 64
    E, K, K2 = _consts()
    M = _mixmat(champion_w, item_w, trait_w)
    return pl.pallas_call(
        _body,
        grid=(B // rb,),
        in_specs=[
            pl.BlockSpec((rb, L, NX), lambda i: (i, 0, 0)),
            pl.BlockSpec((NX, C), lambda i: (0, 0)),
            pl.BlockSpec((C,), lambda i: (0,)),
            pl.BlockSpec((C,), lambda i: (0,)),
            pl.BlockSpec((C, OW), lambda i: (0, 0)),
        ],
        out_specs=pl.BlockSpec((rb, L, OW), lambda i: (i, 0, 0)),
        out_shape=jax.ShapeDtypeStruct((B, L, OW), x.dtype),
        compiler_params=pltpu.CompilerParams(
            dimension_semantics=("arbitrary",),
        ),
    )(x, E, K, K2, M)
